# Initial kernel scaffold; baseline (speedup 1.0000x reference)
import jax
import jax.numpy as jnp
from jax.experimental import pallas as pl


def kernel(hps, ind, inv_mask):
    raise NotImplementedError("write your pallas kernel here")


# speedup vs baseline: 3.1577x; 3.1577x over previous
import jax
import jax.numpy as jnp
from jax import lax
from jax.experimental import pallas as pl
from jax.experimental.pallas import tpu as pltpu
from jax.experimental.pallas import tpu_sc as plsc

_L = 16
_K = 128
_NCH = 8
_B = 64
_HW = 128 * 128


def _body(hps_flat, ind_hbm, mask_hbm, part_hbm, out_hbm,
          idx_v, vals_v, mask_v, part_v, all_v, out_v, sem):
    sid = lax.axis_index("s")
    batches_per_tile = _B // 16

    def batch_body(i, carry):
        acc, msum = carry
        b = sid * batches_per_tile + i
        pltpu.sync_copy(ind_hbm.at[pl.ds(b * _K, _K)], idx_v)
        pltpu.sync_copy(mask_hbm.at[b], mask_v)
        base = b * (16 * _HW)
        copies = []
        for t in range(_K // _L):
            p = idx_v[pl.ds(t * _L, _L)] + base
            for ci in range(_NCH):
                copies.append(pltpu.make_async_copy(
                    hps_flat.at[p + ci * (2 * _HW)],
                    vals_v.at[pl.ds(ci * _K + t * _L, _L)], sem))
        for c in copies:
            c.start()
        for c in copies:
            c.wait()
        for j in range(4):
            for t in range(_K // _L):
                xb = vals_v[pl.ds((j * _K) + t * _L, _L)]
                xu = vals_v[pl.ds(((j + 4) * _K) + t * _L, _L)]
                mb = mask_v[j, pl.ds(t * _L, _L)]
                mu = mask_v[j + 4, pl.ds(t * _L, _L)]
                m = (mb * mu).astype(jnp.float32)
                acc = acc + jnp.abs(xu - xb) * m
                msum = msum + m
        return acc, msum

    zero = jnp.zeros((_L,), jnp.float32)
    acc, msum = lax.fori_loop(0, batches_per_tile, batch_body, (zero, zero))

    part_v[0] = acc
    part_v[1] = msum
    pltpu.sync_copy(part_v, part_hbm.at[sid])
    plsc.subcore_barrier()

    @pl.when(sid == 0)
    def _():
        pltpu.sync_copy(part_hbm, all_v)
        s = jnp.zeros((_L,), jnp.float32)
        m = jnp.zeros((_L,), jnp.float32)
        for i in range(16):
            s = s + all_v[i, 0]
            m = m + all_v[i, 1]
        s_tot = jnp.float32(0)
        m_tot = jnp.float32(0)
        for i in range(_L):
            s_tot = s_tot + s[i]
            m_tot = m_tot + m[i]
        s_vec = jnp.full((_L,), s_tot, jnp.float32)
        m_vec = jnp.full((_L,), m_tot + 0.0001, jnp.float32)
        out_v[...] = s_vec / m_vec
        pltpu.sync_copy(out_v, out_hbm)


@jax.jit
def _kp_align(hps_flat, ind_flat, mask_t):
    mesh = plsc.VectorSubcoreMesh(
        core_axis_name="c", subcore_axis_name="s", num_cores=1)
    fn = pl.kernel(
        _body,
        out_type=(
            jax.ShapeDtypeStruct((16, 2, _L), jnp.float32),
            jax.ShapeDtypeStruct((_L,), jnp.float32),
        ),
        mesh=mesh,
        scratch_types=[
            pltpu.VMEM((_K,), jnp.int32),
            pltpu.VMEM((_NCH * _K,), jnp.float32),
            pltpu.VMEM((_NCH, _K), jnp.int32),
            pltpu.VMEM((2, _L), jnp.float32),
            pltpu.VMEM((16, 2, _L), jnp.float32),
            pltpu.VMEM((_L,), jnp.float32),
            pltpu.SemaphoreType.DMA,
        ],
    )
    return fn(hps_flat, ind_flat, mask_t)


def kernel(hps, ind, inv_mask):
    B, C, H, W = hps.shape
    hps_flat = hps.reshape(B * C * H * W)
    mask_t = jnp.transpose(inv_mask[:, :, 0:16:2], (0, 2, 1))
    _, out = _kp_align(hps_flat, ind.reshape(B * _K), mask_t)
    return out[0]


# speedup vs baseline: 3.4615x; 1.0962x over previous
import jax
import jax.numpy as jnp
from jax import lax
from jax.experimental import pallas as pl
from jax.experimental.pallas import tpu as pltpu
from jax.experimental.pallas import tpu_sc as plsc

_L = 16
_K = 128
_NCH = 8
_B = 64
_BPT = 4
_HW = 128 * 128


def _body(hps_flat, ind_hbm, mask_hbm, part_hbm, out_hbm,
          idx_v, vals_v, mask_v, part_v, all_v, out_v,
          sem_m, sem0, sem1, sem2, sem3):
    sid = lax.axis_index("s")
    gsems = (sem0, sem1, sem2, sem3)

    meta = []
    for bb in range(_BPT):
        b = sid * _BPT + bb
        meta.append(pltpu.make_async_copy(
            ind_hbm.at[pl.ds(b * _K, _K)], idx_v.at[bb], sem_m))
        meta.append(pltpu.make_async_copy(
            mask_hbm.at[b], mask_v.at[bb], sem_m))
    for c in meta:
        c.start()
    for c in meta:
        c.wait()

    gathers = [[] for _ in range(_BPT)]
    for bb in range(_BPT):
        base = (sid * _BPT + bb) * (16 * _HW)
        for t in range(_K // _L):
            p = idx_v[bb, pl.ds(t * _L, _L)] + base
            for ci in range(_NCH):
                gathers[bb].append(pltpu.make_async_copy(
                    hps_flat.at[p + ci * (2 * _HW)],
                    vals_v.at[pl.ds((bb * _NCH + ci) * _K + t * _L, _L)],
                    gsems[bb]))
    for g in gathers:
        for c in g:
            c.start()

    acc = jnp.zeros((_L,), jnp.float32)
    msum = jnp.zeros((_L,), jnp.float32)
    for bb in range(_BPT):
        for c in gathers[bb]:
            c.wait()
        for j in range(4):
            for t in range(_K // _L):
                xb = vals_v[pl.ds((bb * _NCH + j) * _K + t * _L, _L)]
                xu = vals_v[pl.ds((bb * _NCH + j + 4) * _K + t * _L, _L)]
                mb = mask_v[bb, j, pl.ds(t * _L, _L)]
                mu = mask_v[bb, j + 4, pl.ds(t * _L, _L)]
                m = (mb * mu).astype(jnp.float32)
                acc = acc + jnp.abs(xu - xb) * m
                msum = msum + m

    part_v[0] = acc
    part_v[1] = msum
    pltpu.sync_copy(part_v, part_hbm.at[sid])
    plsc.subcore_barrier()

    @pl.when(sid == 0)
    def _():
        pltpu.sync_copy(part_hbm, all_v)
        s = jnp.zeros((_L,), jnp.float32)
        m = jnp.zeros((_L,), jnp.float32)
        for i in range(16):
            s = s + all_v[i, 0]
            m = m + all_v[i, 1]
        s_tot = jnp.float32(0)
        m_tot = jnp.float32(0)
        for i in range(_L):
            s_tot = s_tot + s[i]
            m_tot = m_tot + m[i]
        s_vec = jnp.full((_L,), s_tot, jnp.float32)
        m_vec = jnp.full((_L,), m_tot + 0.0001, jnp.float32)
        out_v[...] = s_vec / m_vec
        pltpu.sync_copy(out_v, out_hbm)


@jax.jit
def _kp_align(hps_flat, ind_flat, mask_t):
    mesh = plsc.VectorSubcoreMesh(
        core_axis_name="c", subcore_axis_name="s", num_cores=1)
    fn = pl.kernel(
        _body,
        out_type=(
            jax.ShapeDtypeStruct((16, 2, _L), jnp.float32),
            jax.ShapeDtypeStruct((_L,), jnp.float32),
        ),
        mesh=mesh,
        scratch_types=[
            pltpu.VMEM((_BPT, _K), jnp.int32),
            pltpu.VMEM((_BPT * _NCH * _K,), jnp.float32),
            pltpu.VMEM((_BPT, _NCH, _K), jnp.int32),
            pltpu.VMEM((2, _L), jnp.float32),
            pltpu.VMEM((16, 2, _L), jnp.float32),
            pltpu.VMEM((_L,), jnp.float32),
            pltpu.SemaphoreType.DMA,
            pltpu.SemaphoreType.DMA,
            pltpu.SemaphoreType.DMA,
            pltpu.SemaphoreType.DMA,
            pltpu.SemaphoreType.DMA,
        ],
    )
    return fn(hps_flat, ind_flat, mask_t)


def kernel(hps, ind, inv_mask):
    B, C, H, W = hps.shape
    hps_flat = hps.reshape(B * C * H * W)
    mask_t = jnp.transpose(inv_mask[:, :, 0:16:2], (0, 2, 1))
    _, out = _kp_align(hps_flat, ind.reshape(B * _K), mask_t)
    return out[0]


# speedup vs baseline: 3.7152x; 1.0733x over previous
import jax
import jax.numpy as jnp
from jax import lax
from jax.experimental import pallas as pl
from jax.experimental.pallas import tpu as pltpu
from jax.experimental.pallas import tpu_sc as plsc

_L = 16
_K = 128
_NCH = 8
_B = 64
_BPT = 4
_HW = 128 * 128


def _body(hps_flat, ind_hbm, mask_hbm, part_hbm, out_hbm,
          idx_v, gidx_v, vals_v, mask_v, part_v, all_v, out_v,
          sem_m, sem0, sem1, sem2, sem3):
    sid = lax.axis_index("s")
    gsems = (sem0, sem1, sem2, sem3)

    c_ind = pltpu.make_async_copy(
        ind_hbm.at[pl.ds(sid * (_BPT * _K), _BPT * _K)], idx_v, sem_m)
    c_msk = pltpu.make_async_copy(
        mask_hbm.at[pl.ds(sid * _BPT, _BPT)], mask_v, sem_m)
    c_ind.start()
    c_msk.start()
    c_ind.wait()
    c_msk.wait()

    for bb in range(_BPT):
        base = (sid * _BPT + bb) * (16 * _HW)
        for t in range(_K // _L):
            p = idx_v[pl.ds(bb * _K + t * _L, _L)] + base
            for ci in range(_NCH):
                gidx_v[bb * _NCH + ci, pl.ds(t * _L, _L)] = \
                    p + ci * (2 * _HW)
    gathers = [[] for _ in range(_BPT)]
    for bb in range(_BPT):
        for ci in range(_NCH):
            r = bb * _NCH + ci
            gathers[bb].append(pltpu.make_async_copy(
                hps_flat.at[gidx_v.at[r]],
                vals_v.at[pl.ds(r * _K, _K)], gsems[bb]))
    for g in gathers:
        for c in g:
            c.start()

    acc = jnp.zeros((_L,), jnp.float32)
    msum = jnp.zeros((_L,), jnp.float32)
    for bb in range(_BPT):
        for c in gathers[bb]:
            c.wait()
        for j in range(4):
            for t in range(_K // _L):
                xb = vals_v[pl.ds((bb * _NCH + j) * _K + t * _L, _L)]
                xu = vals_v[pl.ds((bb * _NCH + j + 4) * _K + t * _L, _L)]
                mb = mask_v[bb, j, pl.ds(t * _L, _L)]
                mu = mask_v[bb, j + 4, pl.ds(t * _L, _L)]
                m = (mb * mu).astype(jnp.float32)
                acc = acc + jnp.abs(xu - xb) * m
                msum = msum + m

    part_v[0] = acc
    part_v[1] = msum
    pltpu.sync_copy(part_v, part_hbm.at[sid])
    plsc.subcore_barrier()

    @pl.when(sid == 0)
    def _():
        pltpu.sync_copy(part_hbm, all_v)
        s = jnp.zeros((_L,), jnp.float32)
        m = jnp.zeros((_L,), jnp.float32)
        for i in range(16):
            s = s + all_v[i, 0]
            m = m + all_v[i, 1]
        s_tot = jnp.float32(0)
        m_tot = jnp.float32(0)
        for i in range(_L):
            s_tot = s_tot + s[i]
            m_tot = m_tot + m[i]
        s_vec = jnp.full((_L,), s_tot, jnp.float32)
        m_vec = jnp.full((_L,), m_tot + 0.0001, jnp.float32)
        out_v[...] = s_vec / m_vec
        pltpu.sync_copy(out_v, out_hbm)


@jax.jit
def _kp_align(hps_flat, ind_flat, mask_t):
    mesh = plsc.VectorSubcoreMesh(
        core_axis_name="c", subcore_axis_name="s", num_cores=1)
    fn = pl.kernel(
        _body,
        out_type=(
            jax.ShapeDtypeStruct((16, 2, _L), jnp.float32),
            jax.ShapeDtypeStruct((_L,), jnp.float32),
        ),
        mesh=mesh,
        scratch_types=[
            pltpu.VMEM((_BPT * _K,), jnp.int32),
            pltpu.VMEM((_BPT * _NCH, _K), jnp.int32),
            pltpu.VMEM((_BPT * _NCH * _K,), jnp.float32),
            pltpu.VMEM((_BPT, _NCH, _K), jnp.int32),
            pltpu.VMEM((2, _L), jnp.float32),
            pltpu.VMEM((16, 2, _L), jnp.float32),
            pltpu.VMEM((_L,), jnp.float32),
            pltpu.SemaphoreType.DMA,
            pltpu.SemaphoreType.DMA,
            pltpu.SemaphoreType.DMA,
            pltpu.SemaphoreType.DMA,
            pltpu.SemaphoreType.DMA,
        ],
    )
    return fn(hps_flat, ind_flat, mask_t)


def kernel(hps, ind, inv_mask):
    B, C, H, W = hps.shape
    hps_flat = hps.reshape(B * C * H * W)
    mask_t = jnp.transpose(inv_mask[:, :, 0:16:2], (0, 2, 1))
    _, out = _kp_align(hps_flat, ind.reshape(B * _K), mask_t)
    return out[0]


# speedup vs baseline: 3.7232x; 1.0022x over previous
import jax
import jax.numpy as jnp
from jax import lax
from jax.experimental import pallas as pl
from jax.experimental.pallas import tpu as pltpu
from jax.experimental.pallas import tpu_sc as plsc

_L = 16
_K = 128
_NCH = 8
_B = 64
_BPT = 4
_HW = 128 * 128


def _body(hps_flat, ind_hbm, mask_hbm, part_hbm, out_hbm,
          idx_v, gidx_v, vals_v, mask_v, part_v, all_v, out_v,
          sem_m, sem0, sem1, sem2, sem3):
    sid = lax.axis_index("s")
    gsems = (sem0, sem1, sem2, sem3)

    c_ind = pltpu.make_async_copy(
        ind_hbm.at[pl.ds(sid * (_BPT * _K), _BPT * _K)], idx_v, sem_m)
    c_msk = pltpu.make_async_copy(
        mask_hbm.at[pl.ds(sid * _BPT, _BPT)], mask_v, sem_m)
    c_ind.start()
    c_msk.start()
    c_ind.wait()
    c_msk.wait()

    gathers = [[] for _ in range(_BPT)]
    for bb in range(_BPT):
        base = (sid * _BPT + bb) * (16 * _HW)
        for t in range(_K // _L):
            p = idx_v[pl.ds(bb * _K + t * _L, _L)] + base
            for ci in range(_NCH):
                gidx_v[bb * _NCH + ci, pl.ds(t * _L, _L)] = \
                    p + ci * (2 * _HW)
        for ci in range(_NCH):
            r = bb * _NCH + ci
            c = pltpu.make_async_copy(
                hps_flat.at[gidx_v.at[r]],
                vals_v.at[pl.ds(r * _K, _K)], gsems[bb])
            c.start()
            gathers[bb].append(c)

    acc = jnp.zeros((_L,), jnp.float32)
    msum = jnp.zeros((_L,), jnp.float32)
    for bb in range(_BPT):
        for c in gathers[bb]:
            c.wait()
        for j in range(4):
            for t in range(_K // _L):
                xb = vals_v[pl.ds((bb * _NCH + j) * _K + t * _L, _L)]
                xu = vals_v[pl.ds((bb * _NCH + j + 4) * _K + t * _L, _L)]
                mb = mask_v[bb, j, pl.ds(t * _L, _L)]
                mu = mask_v[bb, j + 4, pl.ds(t * _L, _L)]
                m = (mb * mu).astype(jnp.float32)
                acc = acc + jnp.abs(xu - xb) * m
                msum = msum + m

    part_v[0] = acc
    part_v[1] = msum
    pltpu.sync_copy(part_v, part_hbm.at[sid])
    plsc.subcore_barrier()

    @pl.when(sid == 0)
    def _():
        pltpu.sync_copy(part_hbm, all_v)
        s = jnp.zeros((_L,), jnp.float32)
        m = jnp.zeros((_L,), jnp.float32)
        for i in range(16):
            s = s + all_v[i, 0]
            m = m + all_v[i, 1]
        s_tot = jnp.float32(0)
        m_tot = jnp.float32(0)
        for i in range(_L):
            s_tot = s_tot + s[i]
            m_tot = m_tot + m[i]
        s_vec = jnp.full((_L,), s_tot, jnp.float32)
        m_vec = jnp.full((_L,), m_tot + 0.0001, jnp.float32)
        out_v[...] = s_vec / m_vec
        pltpu.sync_copy(out_v, out_hbm)


@jax.jit
def _kp_align(hps_flat, ind_flat, mask_t):
    mesh = plsc.VectorSubcoreMesh(
        core_axis_name="c", subcore_axis_name="s", num_cores=1)
    fn = pl.kernel(
        _body,
        out_type=(
            jax.ShapeDtypeStruct((16, 2, _L), jnp.float32),
            jax.ShapeDtypeStruct((_L,), jnp.float32),
        ),
        mesh=mesh,
        scratch_types=[
            pltpu.VMEM((_BPT * _K,), jnp.int32),
            pltpu.VMEM((_BPT * _NCH, _K), jnp.int32),
            pltpu.VMEM((_BPT * _NCH * _K,), jnp.float32),
            pltpu.VMEM((_BPT, _NCH, _K), jnp.int32),
            pltpu.VMEM((2, _L), jnp.float32),
            pltpu.VMEM((16, 2, _L), jnp.float32),
            pltpu.VMEM((_L,), jnp.float32),
            pltpu.SemaphoreType.DMA,
            pltpu.SemaphoreType.DMA,
            pltpu.SemaphoreType.DMA,
            pltpu.SemaphoreType.DMA,
            pltpu.SemaphoreType.DMA,
        ],
    )
    return fn(hps_flat, ind_flat, mask_t)


def kernel(hps, ind, inv_mask):
    B, C, H, W = hps.shape
    hps_flat = hps.reshape(B * C * H * W)
    mask_t = jnp.transpose(inv_mask[:, :, 0:16:2], (0, 2, 1))
    _, out = _kp_align(hps_flat, ind.reshape(B * _K), mask_t)
    return out[0]


# speedup vs baseline: 3.9040x; 1.0486x over previous
import jax
import jax.numpy as jnp
from jax import lax
from jax.experimental import pallas as pl
from jax.experimental.pallas import tpu as pltpu
from jax.experimental.pallas import tpu_sc as plsc

_L = 16
_K = 128
_NCH = 8
_B = 64
_BPT = 2
_HW = 128 * 128


def _body(hps_flat, ind_hbm, mask_hbm, part_hbm, sums_hbm,
          idx_v, gidx_v, vals_v, mask_v, part_v, all_v, out_v,
          sem_m, sem0, sem1):
    cid = lax.axis_index("c")
    sid = lax.axis_index("s")
    wid = cid * 16 + sid
    gsems = (sem0, sem1)

    c_ind = pltpu.make_async_copy(
        ind_hbm.at[pl.ds(wid * (_BPT * _K), _BPT * _K)], idx_v, sem_m)
    c_msk = pltpu.make_async_copy(
        mask_hbm.at[pl.ds(wid * _BPT, _BPT)], mask_v, sem_m)
    c_ind.start()
    c_msk.start()
    c_ind.wait()

    gathers = [[] for _ in range(_BPT)]
    for bb in range(_BPT):
        base = (wid * _BPT + bb) * (16 * _HW)
        for t in range(_K // _L):
            p = idx_v[pl.ds(bb * _K + t * _L, _L)] + base
            for ci in range(_NCH):
                gidx_v[bb * _NCH + ci, pl.ds(t * _L, _L)] = \
                    p + ci * (2 * _HW)
        for ci in range(_NCH):
            r = bb * _NCH + ci
            c = pltpu.make_async_copy(
                hps_flat.at[gidx_v.at[r]],
                vals_v.at[pl.ds(r * _K, _K)], gsems[bb])
            c.start()
            gathers[bb].append(c)
    c_msk.wait()

    acc = jnp.zeros((_L,), jnp.float32)
    msum = jnp.zeros((_L,), jnp.float32)
    for bb in range(_BPT):
        for c in gathers[bb]:
            c.wait()
        for j in range(4):
            for t in range(_K // _L):
                xb = vals_v[pl.ds((bb * _NCH + j) * _K + t * _L, _L)]
                xu = vals_v[pl.ds((bb * _NCH + j + 4) * _K + t * _L, _L)]
                mb = mask_v[bb, j, pl.ds(t * _L, _L)]
                mu = mask_v[bb, j + 4, pl.ds(t * _L, _L)]
                m = (mb * mu).astype(jnp.float32)
                acc = acc + jnp.abs(xu - xb) * m
                msum = msum + m

    part_v[0] = acc
    part_v[1] = msum
    pltpu.sync_copy(part_v, part_hbm.at[cid, sid])
    plsc.subcore_barrier()

    @pl.when(sid == 0)
    def _():
        pltpu.sync_copy(part_hbm.at[cid], all_v)
        s = jnp.zeros((_L,), jnp.float32)
        m = jnp.zeros((_L,), jnp.float32)
        for i in range(16):
            s = s + all_v[i, 0]
            m = m + all_v[i, 1]
        out_v[0] = s
        out_v[1] = m
        pltpu.sync_copy(out_v.at[0], sums_hbm.at[2 * cid, pl.ds(0, _L)])
        pltpu.sync_copy(out_v.at[1], sums_hbm.at[2 * cid + 1, pl.ds(0, _L)])


def _tc_combine(x_ref, o_ref):
    x = x_ref[...]
    lane = lax.broadcasted_iota(jnp.int32, (4, 128), 1)
    x = jnp.where(lane < _L, x, 0.0)
    s = jnp.sum(x[0:1] + x[2:3])
    m = jnp.sum(x[1:2] + x[3:4])
    o_ref[0, 0] = s / (m + 0.0001)


@jax.jit
def _kp_align(hps_flat, ind_flat, mask_t):
    mesh = plsc.VectorSubcoreMesh(
        core_axis_name="c", subcore_axis_name="s", num_cores=2)
    fn = pl.kernel(
        _body,
        out_type=(
            jax.ShapeDtypeStruct((2, 16, 2, _L), jnp.float32),
            jax.ShapeDtypeStruct((4, 128), jnp.float32),
        ),
        mesh=mesh,
        scratch_types=[
            pltpu.VMEM((_BPT * _K,), jnp.int32),
            pltpu.VMEM((_BPT * _NCH, _K), jnp.int32),
            pltpu.VMEM((_BPT * _NCH * _K,), jnp.float32),
            pltpu.VMEM((_BPT, _NCH, _K), jnp.int32),
            pltpu.VMEM((2, _L), jnp.float32),
            pltpu.VMEM((16, 2, _L), jnp.float32),
            pltpu.VMEM((2, _L), jnp.float32),
            pltpu.SemaphoreType.DMA,
            pltpu.SemaphoreType.DMA,
            pltpu.SemaphoreType.DMA,
        ],
    )
    _, sums = fn(hps_flat, ind_flat, mask_t)
    loss = pl.pallas_call(
        _tc_combine,
        out_shape=jax.ShapeDtypeStruct((1, 1), jnp.float32),
        out_specs=pl.BlockSpec(memory_space=pltpu.SMEM),
    )(sums)
    return loss


def kernel(hps, ind, inv_mask):
    B, C, H, W = hps.shape
    hps_flat = hps.reshape(B * C * H * W)
    mask_t = jnp.transpose(inv_mask[:, :, 0:16:2], (0, 2, 1))
    loss = _kp_align(hps_flat, ind.reshape(B * _K), mask_t)
    return loss[0, 0]


# speedup vs baseline: 4.0151x; 1.0284x over previous
import jax
import jax.numpy as jnp
from jax import lax
from jax.experimental import pallas as pl
from jax.experimental.pallas import tpu as pltpu
from jax.experimental.pallas import tpu_sc as plsc

_L = 16
_K = 128
_NCH = 8
_B = 64
_BPT = 2
_HW = 128 * 128


def _body(hps_flat, ind_hbm, mask_hbm, part_hbm,
          idx_v, gidx_v, vals_v, mask_v, part_v,
          sem_m, sem0, sem1):
    cid = lax.axis_index("c")
    sid = lax.axis_index("s")
    wid = cid * 16 + sid
    gsems = (sem0, sem1)

    c_ind = pltpu.make_async_copy(
        ind_hbm.at[pl.ds(wid * (_BPT * _K), _BPT * _K)], idx_v, sem_m)
    c_msk = pltpu.make_async_copy(
        mask_hbm.at[pl.ds(wid * _BPT, _BPT)], mask_v, sem_m)
    c_ind.start()
    c_msk.start()
    c_ind.wait()

    gathers = [[] for _ in range(_BPT)]
    for bb in range(_BPT):
        base = (wid * _BPT + bb) * (16 * _HW)
        for t in range(_K // _L):
            p = idx_v[pl.ds(bb * _K + t * _L, _L)] + base
            for ci in range(_NCH):
                gidx_v[bb * _NCH + ci, pl.ds(t * _L, _L)] = \
                    p + ci * (2 * _HW)
        for ci in range(_NCH):
            r = bb * _NCH + ci
            c = pltpu.make_async_copy(
                hps_flat.at[gidx_v.at[r]],
                vals_v.at[pl.ds(r * _K, _K)], gsems[bb])
            c.start()
            gathers[bb].append(c)
    c_msk.wait()

    acc = jnp.zeros((_L,), jnp.float32)
    msum = jnp.zeros((_L,), jnp.float32)
    for bb in range(_BPT):
        for c in gathers[bb]:
            c.wait()
        for j in range(4):
            for t in range(_K // _L):
                xb = vals_v[pl.ds((bb * _NCH + j) * _K + t * _L, _L)]
                xu = vals_v[pl.ds((bb * _NCH + j + 4) * _K + t * _L, _L)]
                mb = mask_v[bb, j, pl.ds(t * _L, _L)]
                mu = mask_v[bb, j + 4, pl.ds(t * _L, _L)]
                m = (mb * mu).astype(jnp.float32)
                acc = acc + jnp.abs(xu - xb) * m
                msum = msum + m

    part_v[0] = acc
    part_v[1] = msum
    w0 = pltpu.make_async_copy(
        part_v.at[0], part_hbm.at[wid, 0, pl.ds(0, _L)], sem_m)
    w1 = pltpu.make_async_copy(
        part_v.at[1], part_hbm.at[wid, 1, pl.ds(0, _L)], sem_m)
    w0.start()
    w1.start()
    w0.wait()
    w1.wait()


def _tc_combine(x_ref, o_ref):
    x = x_ref[...]
    lane = lax.broadcasted_iota(jnp.int32, (32, 2, 128), 2)
    x = jnp.where(lane < _L, x, 0.0)
    s = jnp.sum(x[:, 0, :])
    m = jnp.sum(x[:, 1, :])
    o_ref[0, 0] = s / (m + 0.0001)


@jax.jit
def _kp_align(hps_flat, ind_flat, mask_t):
    mesh = plsc.VectorSubcoreMesh(
        core_axis_name="c", subcore_axis_name="s", num_cores=2)
    fn = pl.kernel(
        _body,
        out_type=jax.ShapeDtypeStruct((32, 2, 128), jnp.float32),
        mesh=mesh,
        scratch_types=[
            pltpu.VMEM((_BPT * _K,), jnp.int32),
            pltpu.VMEM((_BPT * _NCH, _K), jnp.int32),
            pltpu.VMEM((_BPT * _NCH * _K,), jnp.float32),
            pltpu.VMEM((_BPT, _NCH, _K), jnp.int32),
            pltpu.VMEM((2, _L), jnp.float32),
            pltpu.SemaphoreType.DMA,
            pltpu.SemaphoreType.DMA,
            pltpu.SemaphoreType.DMA,
        ],
    )
    parts = fn(hps_flat, ind_flat, mask_t)
    loss = pl.pallas_call(
        _tc_combine,
        out_shape=jax.ShapeDtypeStruct((1, 1), jnp.float32),
        out_specs=pl.BlockSpec(memory_space=pltpu.SMEM),
    )(parts)
    return loss


def kernel(hps, ind, inv_mask):
    B, C, H, W = hps.shape
    hps_flat = hps.reshape(B * C * H * W)
    mask_t = jnp.transpose(inv_mask[:, :, 0:16:2], (0, 2, 1))
    loss = _kp_align(hps_flat, ind.reshape(B * _K), mask_t)
    return loss[0, 0]
